# baseline (speedup 1.0000x reference)
import jax
import jax.numpy as jnp
from jax.experimental import pallas as pl
from jax.experimental.pallas import tpu as pltpu
from jax.experimental.pallas import tpu_sc as plsc

PRE_SEQ_LEN = 128
BATCH = 16
EMB_DIM = 18432
N_ROWS = BATCH * PRE_SEQ_LEN
CHUNK = 384
N_CHUNKS = EMB_DIM // CHUNK
N_GROWS = N_ROWS * N_CHUNKS
W = 128

_MESH = plsc.VectorSubcoreMesh(core_axis_name="core", subcore_axis_name="subcore")


def kernel(prefix, embedding_table):
    flat_idx = prefix.reshape(N_ROWS).astype(jnp.int32)
    exp_idx = (
        flat_idx[:, None] * N_CHUNKS + jnp.arange(N_CHUNKS, dtype=jnp.int32)[None, :]
    ).reshape(1, N_GROWS)
    tbl = embedding_table.reshape(PRE_SEQ_LEN * N_CHUNKS, CHUNK)

    @pl.kernel(
        out_type=jax.ShapeDtypeStruct((N_GROWS, CHUNK), jnp.float32),
        mesh=_MESH,
    )
    def sc_gather(tbl_hbm, idx_hbm, out_hbm):
        def body(i_vmem, o_vmem):
            pltpu.sync_copy(tbl_hbm.at[i_vmem.at[0]], o_vmem)

        pltpu.emit_pipeline(
            body,
            grid=(N_GROWS // W,),
            in_specs=[pl.BlockSpec((1, W), lambda i: (0, i))],
            out_specs=[pl.BlockSpec((W, CHUNK), lambda i: (i, 0))],
            core_axis_name=("core", "subcore"),
            dimension_semantics=(pltpu.PARALLEL,),
        )(idx_hbm, out_hbm)

    out = sc_gather(tbl, exp_idx)
    return out.reshape(BATCH, PRE_SEQ_LEN, EMB_DIM)


# speedup vs baseline: 2.2183x; 2.2183x over previous
import jax
import jax.numpy as jnp
from jax.experimental import pallas as pl
from jax.experimental.pallas import tpu as pltpu
from jax.experimental.pallas import tpu_sc as plsc

PRE_SEQ_LEN = 128
BATCH = 16
EMB_DIM = 18432
N_ROWS = BATCH * PRE_SEQ_LEN
W = 2
N_STEPS = N_ROWS // W

_MESH = plsc.VectorSubcoreMesh(core_axis_name="core", subcore_axis_name="subcore")


def kernel(prefix, embedding_table):
    idx_blocks = jnp.pad(prefix.reshape(N_STEPS, W), ((0, 0), (0, 128 - W)))

    @pl.kernel(
        out_type=jax.ShapeDtypeStruct((N_ROWS, EMB_DIM), jnp.float32),
        mesh=_MESH,
    )
    def sc_gather(tbl_hbm, idx_hbm, out_hbm):
        def body(i_vmem, o_vmem):
            pltpu.sync_copy(tbl_hbm.at[i_vmem.at[0, pl.ds(0, W)]], o_vmem)

        pltpu.emit_pipeline(
            body,
            grid=(N_STEPS,),
            in_specs=[pl.BlockSpec((1, 128), lambda i: (i, 0))],
            out_specs=[pl.BlockSpec((W, EMB_DIM), lambda i: (i, 0))],
            core_axis_name=("core", "subcore"),
            dimension_semantics=(pltpu.PARALLEL,),
        )(idx_hbm, out_hbm)

    out = sc_gather(embedding_table, idx_blocks)
    return out.reshape(BATCH, PRE_SEQ_LEN, EMB_DIM)


# speedup vs baseline: 2.2718x; 1.0241x over previous
import functools

import jax
import jax.numpy as jnp
from jax import lax
from jax.experimental import pallas as pl
from jax.experimental.pallas import tpu as pltpu
from jax.experimental.pallas import tpu_sc as plsc

PRE_SEQ_LEN = 128
BATCH = 16
EMB_DIM = 18432
N_ROWS = BATCH * PRE_SEQ_LEN
NW = 32
ROWS_PER_W = N_ROWS // NW
NBUF = 4

_MESH = plsc.VectorSubcoreMesh(core_axis_name="core", subcore_axis_name="subcore")


@functools.partial(
    pl.kernel,
    mesh=_MESH,
    out_type=jax.ShapeDtypeStruct((N_ROWS, EMB_DIM), jnp.float32),
    scratch_types=[
        pltpu.VMEM((1, 128), jnp.int32),
        pltpu.VMEM((NBUF, 1, EMB_DIM), jnp.float32),
        pltpu.SemaphoreType.DMA,
        pltpu.SemaphoreType.DMA((NBUF,)),
        pltpu.SemaphoreType.DMA((NBUF,)),
    ],
)
def _sc_gather(tbl_hbm, idx_hbm, out_hbm, idx_v, bufs, sem_i, sem_g, sem_o):
    wid = lax.axis_index("subcore") * 2 + lax.axis_index("core")
    base = wid * ROWS_PER_W

    cp_i = pltpu.make_async_copy(idx_hbm.at[pl.ds(wid, 1)], idx_v, sem_i)
    cp_i.start()
    cp_i.wait()

    def gather(g, b):
        return pltpu.make_async_copy(
            tbl_hbm.at[idx_v.at[0, pl.ds(g, 1)]], bufs.at[b], sem_g.at[b]
        )

    def put(g, b):
        return pltpu.make_async_copy(
            bufs.at[b], out_hbm.at[pl.ds(base + g, 1)], sem_o.at[b]
        )

    for b in range(NBUF - 1):
        gather(b, b).start()

    @pl.loop(0, ROWS_PER_W, step=NBUF)
    def _(g0):
        for j in range(NBUF):
            g = g0 + j
            gather(g, j).wait()
            put(g, j).start()
            nxt = g + NBUF - 1
            bn = (j + NBUF - 1) % NBUF
            prev = g - 1

            @pl.when(nxt < ROWS_PER_W)
            def _():
                @pl.when(prev >= 0)
                def _():
                    put(prev, bn).wait()

                gather(nxt, bn).start()

    for j in range(NBUF):
        put(ROWS_PER_W - NBUF + j, j).wait()


def kernel(prefix, embedding_table):
    idx_blocks = jnp.pad(
        prefix.reshape(NW, ROWS_PER_W), ((0, 0), (0, 128 - ROWS_PER_W))
    )
    out = _sc_gather(embedding_table, idx_blocks)
    return out.reshape(BATCH, PRE_SEQ_LEN, EMB_DIM)
